# baseline (speedup 1.0000x reference)
import functools

import jax
import jax.numpy as jnp
from jax import lax
from jax.experimental import pallas as pl
from jax.experimental.pallas import tpu as pltpu
from jax.experimental.pallas import tpu_sc as plsc

N = 10000
D = 256
E = 160000
ETYPES = 3

CHUNK = 128
TPT = 80
EPT = TPT * CHUNK
NE_PAD = EPT * 16
ROWS2D = NE_PAD // CHUNK
NPAD = 10240
RPT = NPAD // 16
SENT = N
HTPT = TPT // 2
BM = 2000

@functools.cache
def _mesh():
    return plsc.VectorSubcoreMesh(core_axis_name="c", subcore_axis_name="s")


def _f32(shape):
    return jax.ShapeDtypeStruct(shape, jnp.float32)


HR = NPAD // 128
HRT = HR // 10


def _hist_body(hidx, z128, deg, shred, idxv, histbuf, redbuf, sem):
    s = lax.axis_index("s")
    c = lax.axis_index("c")
    ones16 = jnp.ones((16,), jnp.float32)
    for e in range(ETYPES):
        for cc in range(2):
            q = cc * 3 + e

            @pl.when(c == cc)
            def _(q=q):
                pltpu.sync_copy(z128.at[pl.ds(0, HR)], histbuf)
                pltpu.sync_copy(hidx.at[q, pl.ds(s * TPT, TPT)], idxv)

                def row(j, carry):
                    for k in range(8):
                        v = idxv[j, pl.ds(k * 16, 16)]
                        rows = jnp.right_shift(v, 7)
                        cols = jnp.bitwise_and(v, 127)
                        plsc.addupdate_scatter(histbuf, [rows, cols], ones16)
                    return carry

                lax.fori_loop(0, TPT, row, 0)
                pltpu.sync_copy(histbuf, shred.at[s])

        plsc.subcore_barrier()
        for cc in range(2):
            q = cc * 3 + e

            @pl.when((c == cc) & (s < 10))
            def _(q=q):
                pltpu.sync_copy(shred.at[:, pl.ds(s * HRT, HRT), :], redbuf)

                def red(r, carry):
                    for k in range(8):
                        acc = redbuf[0, r, pl.ds(k * 16, 16)]
                        for t in range(1, 16):
                            acc = acc + redbuf[t, r, pl.ds(k * 16, 16)]
                        histbuf[r, pl.ds(k * 16, 16)] = acc
                    return carry

                lax.fori_loop(0, HRT, red, 0)
                pltpu.sync_copy(histbuf.at[pl.ds(0, HRT)],
                                deg.at[q, pl.ds(s * HRT, HRT)])
        plsc.subcore_barrier()


@functools.cache
def _hist_kernel():
    return pl.kernel(
        _hist_body,
        out_type=[_f32((6, HR, 128))],
        mesh=_mesh(),
        scratch_types=[
            pltpu.VMEM_SHARED((16, HR, 128), jnp.float32),
            pltpu.VMEM((TPT, CHUNK), jnp.int32),
            pltpu.VMEM((HR, 128), jnp.float32),
            pltpu.VMEM((16, HRT, 128), jnp.float32),
            pltpu.SemaphoreType.DMA,
        ],
        compiler_params=pltpu.CompilerParams(needs_layout_passes=False),
    )


def _agg_body(t00, t01, t10, t11, t20, t21, srcg, dstg, z128,
              o00, o01, o10, o11, o20, o21,
              sh, sidx, didx, rb0, rb1, gs0, gs1, ss0, ss1):
    s = lax.axis_index("s")
    c = lax.axis_index("c")
    tables = ((t00, t01), (t10, t11), (t20, t21))
    outs = ((o00, o01), (o10, o11), (o20, o21))
    for e in range(ETYPES):
        for k in range(RPT // CHUNK):
            pltpu.sync_copy(z128, sh.at[pl.ds(s * RPT + k * CHUNK, CHUNK)])
        plsc.subcore_barrier()
        for half in range(2):
            off = s * TPT + half * HTPT
            pltpu.sync_copy(srcg.at[e, pl.ds(off, HTPT)], sidx)
            pltpu.sync_copy(dstg.at[e, pl.ds(off, HTPT)], didx)
            for cc in range(2):

                @pl.when(c == cc)
                def _(e=e, cc=cc):
                    tbl = tables[e][cc]

                    def g(j, buf, sem):
                        return pltpu.make_async_copy(
                            tbl.at[sidx.at[j]], buf, sem)

                    def sc(j, buf, sem):
                        return pltpu.make_async_copy(
                            buf, sh.at[didx.at[j]], sem)

                    g(0, rb0, gs0).start()
                    g(1, rb1, gs1).start()

                    def body(jj, carry):
                        j = 2 * jj
                        g(j, rb0, gs0).wait()
                        sc(j, rb0, ss0).start(add=True)
                        g(j + 1, rb1, gs1).wait()
                        sc(j + 1, rb1, ss1).start(add=True)
                        sc(j, rb0, ss0).wait()
                        g(j + 2, rb0, gs0).start()
                        sc(j + 1, rb1, ss1).wait()
                        g(j + 3, rb1, gs1).start()
                        return carry

                    lax.fori_loop(0, HTPT // 2 - 1, body, 0)
                    j = HTPT - 2
                    g(j, rb0, gs0).wait()
                    sc(j, rb0, ss0).start(add=True)
                    g(j + 1, rb1, gs1).wait()
                    sc(j + 1, rb1, ss1).start(add=True)
                    sc(j, rb0, ss0).wait()
                    sc(j + 1, rb1, ss1).wait()

        plsc.subcore_barrier()
        for cc in range(2):

            @pl.when(c == cc)
            def _(e=e, cc=cc):
                out = outs[e][cc]

                @pl.when(s < 15)
                def _():
                    pltpu.sync_copy(sh.at[pl.ds(s * RPT, RPT)],
                                    out.at[pl.ds(s * RPT, RPT)])

                @pl.when(s == 15)
                def _():
                    pltpu.sync_copy(sh.at[pl.ds(15 * RPT, N - 15 * RPT)],
                                    out.at[pl.ds(15 * RPT, N - 15 * RPT)])


@functools.cache
def _agg_kernel():
    return pl.kernel(
        _agg_body,
        out_type=[_f32((N, 128))] * 6,
        mesh=_mesh(),
        scratch_types=[
            pltpu.VMEM_SHARED((NPAD, 128), jnp.float32),
            pltpu.VMEM((HTPT, CHUNK), jnp.int32),
            pltpu.VMEM((HTPT, CHUNK), jnp.int32),
            pltpu.VMEM((CHUNK, 128), jnp.float32),
            pltpu.VMEM((CHUNK, 128), jnp.float32),
            pltpu.SemaphoreType.DMA,
            pltpu.SemaphoreType.DMA,
            pltpu.SemaphoreType.DMA,
            pltpu.SemaphoreType.DMA,
        ],
    )


def _mm_stats_body(x_ref, w_ref, b_ref, y_ref, st_ref):
    i = pl.program_id(0)
    y = jnp.dot(x_ref[...], w_ref[...], preferred_element_type=jnp.float32)
    y = y + b_ref[...]
    y_ref[...] = y

    @pl.when(i == 0)
    def _():
        st_ref[...] = jnp.zeros_like(st_ref)

    st_ref[0:1, :] = st_ref[0:1, :] + jnp.sum(y, axis=0, keepdims=True)
    st_ref[1:2, :] = st_ref[1:2, :] + jnp.sum(y * y, axis=0, keepdims=True)


def _mm_stats(x, w, b):
    return pl.pallas_call(
        _mm_stats_body,
        grid=(N // BM,),
        in_specs=[
            pl.BlockSpec((BM, D), lambda i: (i, 0)),
            pl.BlockSpec((D, D), lambda i: (0, 0)),
            pl.BlockSpec((1, D), lambda i: (0, 0)),
        ],
        out_specs=[
            pl.BlockSpec((BM, D), lambda i: (i, 0)),
            pl.BlockSpec((8, D), lambda i: (0, 0)),
        ],
        out_shape=[_f32((N, D)), _f32((8, D))],
    )(x, w, b)


def _scales_body(deg_ref, out_ref):
    out_ref[...] = lax.rsqrt(jnp.maximum(deg_ref[...], 1.0))


def _scales(deg):
    return pl.pallas_call(
        _scales_body,
        grid=(1,),
        in_specs=[pl.BlockSpec((6, HR, 128), lambda i: (0, 0, 0))],
        out_specs=pl.BlockSpec((6, HR, 128), lambda i: (0, 0, 0)),
        out_shape=_f32((6, HR, 128)),
    )(deg)


def _norm_body(y_ref, st_ref, gb_ref, sc_ref, h_ref, *hs_refs, leaky):
    st = st_ref[...]
    m = st[0:1, :] / N
    v = st[1:2, :] / N - m * m
    rstd = lax.rsqrt(v + 1e-5)
    h = (y_ref[...] - m) * rstd * gb_ref[0:1, :] + gb_ref[1:2, :]
    if leaky:
        h = jnp.where(h > 0, h, 0.01 * h)
    else:
        h = jnp.maximum(h, 0.0)
    h_ref[...] = h
    if hs_refs:
        sc = sc_ref[...]
        for e in range(ETYPES):
            hs = h * sc[e, :, 0:1]
            hs_refs[2 * e][...] = hs[:, :128]
            hs_refs[2 * e + 1][...] = hs[:, 128:]


def _norm(y, st, gb, scales, leaky, emit_tables):
    n_hs = 6 if emit_tables else 0
    out_shape = [_f32((N, D))] + [_f32((N, 128))] * n_hs
    out_specs = [pl.BlockSpec((BM, D), lambda i: (i, 0))] + [
        pl.BlockSpec((BM, 128), lambda i: (i, 0)) for _ in range(n_hs)
    ]
    return pl.pallas_call(
        functools.partial(_norm_body, leaky=leaky),
        grid=(N // BM,),
        in_specs=[
            pl.BlockSpec((BM, D), lambda i: (i, 0)),
            pl.BlockSpec((8, D), lambda i: (0, 0)),
            pl.BlockSpec((2, D), lambda i: (0, 0)),
            pl.BlockSpec((6, BM, 1), lambda i: (0, i, 0)),
        ],
        out_specs=out_specs,
        out_shape=out_shape,
    )(y, st, gb, scales)


def _combine_body(a00, a01, a10, a11, a20, a21, sc_ref, h_ref, wc_ref, b_ref,
                  y_ref, st_ref):
    i = pl.program_id(0)
    aggs = ((a00, a01), (a10, a11), (a20, a21))
    wc = wc_ref[...]
    sc = sc_ref[...]
    acc = jnp.dot(h_ref[...], wc[3 * D:, :], preferred_element_type=jnp.float32)
    for e in range(ETYPES):
        sd = sc[3 + e, :, 0:1]
        for cc in range(2):
            a = aggs[e][cc][...] * sd
            w = wc[e * D + cc * 128:e * D + (cc + 1) * 128, :]
            acc = acc + jnp.dot(a, w, preferred_element_type=jnp.float32)
    y = acc + b_ref[...]
    y_ref[...] = y

    @pl.when(i == 0)
    def _():
        st_ref[...] = jnp.zeros_like(st_ref)

    st_ref[0:1, :] = st_ref[0:1, :] + jnp.sum(y, axis=0, keepdims=True)
    st_ref[1:2, :] = st_ref[1:2, :] + jnp.sum(y * y, axis=0, keepdims=True)


def _combine(aggs, scales, h, wcat, bsum):
    return pl.pallas_call(
        _combine_body,
        grid=(N // BM,),
        in_specs=[pl.BlockSpec((BM, 128), lambda i: (i, 0)) for _ in range(6)]
        + [
            pl.BlockSpec((6, BM, 1), lambda i: (0, i, 0)),
            pl.BlockSpec((BM, D), lambda i: (i, 0)),
            pl.BlockSpec((4 * D, D), lambda i: (0, 0)),
            pl.BlockSpec((1, D), lambda i: (0, 0)),
        ],
        out_specs=[
            pl.BlockSpec((BM, D), lambda i: (i, 0)),
            pl.BlockSpec((8, D), lambda i: (0, 0)),
        ],
        out_shape=[_f32((N, D)), _f32((8, D))],
    )(*aggs, scales, h, wcat, bsum)


def kernel(x, edge_index_e0, edge_index_e1, edge_index_e2, params):
    edges = [edge_index_e0, edge_index_e1, edge_index_e2]
    pad = NE_PAD - E

    def shape_idx(v, fill):
        return jnp.pad(v, (0, pad), constant_values=fill).reshape(ROWS2D, CHUNK)

    srcg = jnp.stack([shape_idx(ei[0], 0) for ei in edges])
    dstg = jnp.stack([shape_idx(ei[1], SENT) for ei in edges])
    hidx = jnp.stack(
        [shape_idx(ei[0], SENT) for ei in edges]
        + [shape_idx(ei[1], SENT) for ei in edges]
    )
    z128 = jnp.zeros((CHUNK, 128), jnp.float32)

    (deg,) = _hist_kernel()(hidx, z128)
    scales = _scales(deg).reshape(6, NPAD, 1)

    p = params
    y, st = _mm_stats(x, p['W_fr'], p['b_fr'].reshape(1, D))
    gb = jnp.stack([p['g_fr'], p['be_fr']])
    h_and_tables = _norm(y, st, gb, scales, leaky=False, emit_tables=True)
    h, tables = h_and_tables[0], h_and_tables[1:]

    n_layers = len(p['layers'])
    for li, lp in enumerate(p['layers']):
        aggs = _agg_kernel()(*tables, srcg, dstg, z128)
        wcat = jnp.concatenate(
            [w / ETYPES for w in lp['W_gcn']] + [lp['W_sk']], axis=0)
        bsum = (sum(lp['b_gcn']) / ETYPES + lp['b_sk']).reshape(1, D)
        y, st = _combine(aggs, scales, h, wcat, bsum)
        gb = jnp.stack([lp['g'], lp['be']])
        last = li == n_layers - 1
        res = _norm(y, st, gb, scales, leaky=True, emit_tables=not last)
        h, tables = res[0], res[1:]
    return h


# speedup vs baseline: 1.0241x; 1.0241x over previous
import functools

import jax
import jax.numpy as jnp
from jax import lax
from jax.experimental import pallas as pl
from jax.experimental.pallas import tpu as pltpu
from jax.experimental.pallas import tpu_sc as plsc

N = 10000
D = 256
E = 160000
ETYPES = 3

CHUNK = 128
TPT = 80
EPT = TPT * CHUNK
NE_PAD = EPT * 16
ROWS2D = NE_PAD // CHUNK
NPAD = 10240
RPT = NPAD // 16
SENT = N
HTPT = TPT // 2
BM = 2000

@functools.cache
def _mesh():
    return plsc.VectorSubcoreMesh(core_axis_name="c", subcore_axis_name="s")


def _f32(shape):
    return jax.ShapeDtypeStruct(shape, jnp.float32)


def _hist_body(hidx, ones128, z128, deg, shsm, idxv, onesv, sem):
    s = lax.axis_index("s")
    c = lax.axis_index("c")
    pltpu.sync_copy(ones128, onesv)
    for e in range(ETYPES):
        for k in range(RPT // CHUNK):
            pltpu.sync_copy(z128, shsm.at[pl.ds(s * RPT + k * CHUNK, CHUNK)])
        plsc.subcore_barrier()
        for cc in range(2):
            q = cc * 3 + e

            @pl.when(c == cc)
            def _(q=q):
                pltpu.sync_copy(hidx.at[q, pl.ds(s * TPT, TPT)], idxv)

                def chunk(j, carry):
                    pltpu.make_async_copy(
                        onesv, shsm.at[idxv.at[j]], sem).start(add=True)
                    return carry

                lax.fori_loop(0, TPT, chunk, 0)

                def drain(j, carry):
                    pltpu.make_async_copy(
                        onesv, shsm.at[idxv.at[0]], sem).wait()
                    return carry

                lax.fori_loop(0, TPT, drain, 0)

        plsc.subcore_barrier()
        for cc in range(2):
            q = cc * 3 + e

            @pl.when(c == cc)
            def _(q=q):
                pltpu.sync_copy(
                    shsm.at[pl.ds(s * RPT, RPT)],
                    deg.at[q, pl.ds(s * RPT, RPT)],
                )
        plsc.subcore_barrier()


@functools.cache
def _hist_kernel():
    return pl.kernel(
        _hist_body,
        out_type=[_f32((6, NPAD, 128))],
        mesh=_mesh(),
        scratch_types=[
            pltpu.VMEM_SHARED((NPAD, 128), jnp.float32),
            pltpu.VMEM((TPT, CHUNK), jnp.int32),
            pltpu.VMEM((CHUNK, 128), jnp.float32),
            pltpu.SemaphoreType.DMA,
        ],
    )


def _agg_body(t00, t01, t10, t11, t20, t21, srcg, dstg, z128,
              o00, o01, o10, o11, o20, o21,
              sh, sidx, didx, rb0, rb1, gs0, gs1, ss0, ss1):
    s = lax.axis_index("s")
    c = lax.axis_index("c")
    tables = ((t00, t01), (t10, t11), (t20, t21))
    outs = ((o00, o01), (o10, o11), (o20, o21))
    for e in range(ETYPES):
        for k in range(RPT // CHUNK):
            pltpu.sync_copy(z128, sh.at[pl.ds(s * RPT + k * CHUNK, CHUNK)])
        plsc.subcore_barrier()
        for half in range(2):
            off = s * TPT + half * HTPT
            pltpu.sync_copy(srcg.at[e, pl.ds(off, HTPT)], sidx)
            pltpu.sync_copy(dstg.at[e, pl.ds(off, HTPT)], didx)
            for cc in range(2):

                @pl.when(c == cc)
                def _(e=e, cc=cc):
                    tbl = tables[e][cc]

                    def g(j, buf, sem):
                        return pltpu.make_async_copy(
                            tbl.at[sidx.at[j]], buf, sem)

                    def sc(j, buf, sem):
                        return pltpu.make_async_copy(
                            buf, sh.at[didx.at[j]], sem)

                    g(0, rb0, gs0).start()
                    g(1, rb1, gs1).start()

                    def body(jj, carry):
                        j = 2 * jj
                        g(j, rb0, gs0).wait()
                        sc(j, rb0, ss0).start(add=True)
                        g(j + 1, rb1, gs1).wait()
                        sc(j + 1, rb1, ss1).start(add=True)
                        sc(j, rb0, ss0).wait()
                        g(j + 2, rb0, gs0).start()
                        sc(j + 1, rb1, ss1).wait()
                        g(j + 3, rb1, gs1).start()
                        return carry

                    lax.fori_loop(0, HTPT // 2 - 1, body, 0)
                    j = HTPT - 2
                    g(j, rb0, gs0).wait()
                    sc(j, rb0, ss0).start(add=True)
                    g(j + 1, rb1, gs1).wait()
                    sc(j + 1, rb1, ss1).start(add=True)
                    sc(j, rb0, ss0).wait()
                    sc(j + 1, rb1, ss1).wait()

        plsc.subcore_barrier()
        for cc in range(2):

            @pl.when(c == cc)
            def _(e=e, cc=cc):
                out = outs[e][cc]

                @pl.when(s < 15)
                def _():
                    pltpu.sync_copy(sh.at[pl.ds(s * RPT, RPT)],
                                    out.at[pl.ds(s * RPT, RPT)])

                @pl.when(s == 15)
                def _():
                    pltpu.sync_copy(sh.at[pl.ds(15 * RPT, N - 15 * RPT)],
                                    out.at[pl.ds(15 * RPT, N - 15 * RPT)])


@functools.cache
def _agg_kernel():
    return pl.kernel(
        _agg_body,
        out_type=[_f32((N, 128))] * 6,
        mesh=_mesh(),
        scratch_types=[
            pltpu.VMEM_SHARED((NPAD, 128), jnp.float32),
            pltpu.VMEM((HTPT, CHUNK), jnp.int32),
            pltpu.VMEM((HTPT, CHUNK), jnp.int32),
            pltpu.VMEM((CHUNK, 128), jnp.float32),
            pltpu.VMEM((CHUNK, 128), jnp.float32),
            pltpu.SemaphoreType.DMA,
            pltpu.SemaphoreType.DMA,
            pltpu.SemaphoreType.DMA,
            pltpu.SemaphoreType.DMA,
        ],
    )


def _mm_stats_body(x_ref, w_ref, b_ref, y_ref, st_ref):
    i = pl.program_id(0)
    y = jnp.dot(x_ref[...], w_ref[...], preferred_element_type=jnp.float32)
    y = y + b_ref[...]
    y_ref[...] = y

    @pl.when(i == 0)
    def _():
        st_ref[...] = jnp.zeros_like(st_ref)

    st_ref[0:1, :] = st_ref[0:1, :] + jnp.sum(y, axis=0, keepdims=True)
    st_ref[1:2, :] = st_ref[1:2, :] + jnp.sum(y * y, axis=0, keepdims=True)


def _mm_stats(x, w, b):
    return pl.pallas_call(
        _mm_stats_body,
        grid=(N // BM,),
        in_specs=[
            pl.BlockSpec((BM, D), lambda i: (i, 0)),
            pl.BlockSpec((D, D), lambda i: (0, 0)),
            pl.BlockSpec((1, D), lambda i: (0, 0)),
        ],
        out_specs=[
            pl.BlockSpec((BM, D), lambda i: (i, 0)),
            pl.BlockSpec((8, D), lambda i: (0, 0)),
        ],
        out_shape=[_f32((N, D)), _f32((8, D))],
    )(x, w, b)


def _scales_body(deg_ref, out_ref):
    out_ref[...] = lax.rsqrt(jnp.maximum(deg_ref[:, :, 0:16], 1.0))


def _scales(deg):
    return pl.pallas_call(
        _scales_body,
        grid=(NPAD // 1024,),
        in_specs=[pl.BlockSpec((6, 1024, 128), lambda i: (0, i, 0))],
        out_specs=pl.BlockSpec((6, 1024, 16), lambda i: (0, i, 0)),
        out_shape=_f32((6, NPAD, 16)),
    )(deg)


def _norm_body(y_ref, st_ref, gb_ref, sc_ref, h_ref, *hs_refs, leaky):
    st = st_ref[...]
    m = st[0:1, :] / N
    v = st[1:2, :] / N - m * m
    rstd = lax.rsqrt(v + 1e-5)
    h = (y_ref[...] - m) * rstd * gb_ref[0:1, :] + gb_ref[1:2, :]
    if leaky:
        h = jnp.where(h > 0, h, 0.01 * h)
    else:
        h = jnp.maximum(h, 0.0)
    h_ref[...] = h
    if hs_refs:
        sc = sc_ref[...]
        for e in range(ETYPES):
            hs = h * sc[e, :, 0:1]
            hs_refs[2 * e][...] = hs[:, :128]
            hs_refs[2 * e + 1][...] = hs[:, 128:]


def _norm(y, st, gb, scales, leaky, emit_tables):
    n_hs = 6 if emit_tables else 0
    out_shape = [_f32((N, D))] + [_f32((N, 128))] * n_hs
    out_specs = [pl.BlockSpec((BM, D), lambda i: (i, 0))] + [
        pl.BlockSpec((BM, 128), lambda i: (i, 0)) for _ in range(n_hs)
    ]
    return pl.pallas_call(
        functools.partial(_norm_body, leaky=leaky),
        grid=(N // BM,),
        in_specs=[
            pl.BlockSpec((BM, D), lambda i: (i, 0)),
            pl.BlockSpec((8, D), lambda i: (0, 0)),
            pl.BlockSpec((2, D), lambda i: (0, 0)),
            pl.BlockSpec((6, BM, 16), lambda i: (0, i, 0)),
        ],
        out_specs=out_specs,
        out_shape=out_shape,
    )(y, st, gb, scales)


def _combine_body(a00, a01, a10, a11, a20, a21, sc_ref, h_ref, wc_ref, b_ref,
                  y_ref, st_ref):
    i = pl.program_id(0)
    aggs = ((a00, a01), (a10, a11), (a20, a21))
    wc = wc_ref[...]
    sc = sc_ref[...]
    acc = jnp.dot(h_ref[...], wc[3 * D:, :], preferred_element_type=jnp.float32)
    for e in range(ETYPES):
        sd = sc[3 + e, :, 0:1]
        for cc in range(2):
            a = aggs[e][cc][...] * sd
            w = wc[e * D + cc * 128:e * D + (cc + 1) * 128, :]
            acc = acc + jnp.dot(a, w, preferred_element_type=jnp.float32)
    y = acc + b_ref[...]
    y_ref[...] = y

    @pl.when(i == 0)
    def _():
        st_ref[...] = jnp.zeros_like(st_ref)

    st_ref[0:1, :] = st_ref[0:1, :] + jnp.sum(y, axis=0, keepdims=True)
    st_ref[1:2, :] = st_ref[1:2, :] + jnp.sum(y * y, axis=0, keepdims=True)


def _combine(aggs, scales, h, wcat, bsum):
    return pl.pallas_call(
        _combine_body,
        grid=(N // BM,),
        in_specs=[pl.BlockSpec((BM, 128), lambda i: (i, 0)) for _ in range(6)]
        + [
            pl.BlockSpec((6, BM, 16), lambda i: (0, i, 0)),
            pl.BlockSpec((BM, D), lambda i: (i, 0)),
            pl.BlockSpec((4 * D, D), lambda i: (0, 0)),
            pl.BlockSpec((1, D), lambda i: (0, 0)),
        ],
        out_specs=[
            pl.BlockSpec((BM, D), lambda i: (i, 0)),
            pl.BlockSpec((8, D), lambda i: (0, 0)),
        ],
        out_shape=[_f32((N, D)), _f32((8, D))],
    )(*aggs, scales, h, wcat, bsum)


def kernel(x, edge_index_e0, edge_index_e1, edge_index_e2, params):
    edges = [edge_index_e0, edge_index_e1, edge_index_e2]
    pad = NE_PAD - E

    def shape_idx(v, fill):
        return jnp.pad(v, (0, pad), constant_values=fill).reshape(ROWS2D, CHUNK)

    srcg = jnp.stack([shape_idx(ei[0], 0) for ei in edges])
    dstg = jnp.stack([shape_idx(ei[1], SENT) for ei in edges])
    hidx = jnp.stack(
        [shape_idx(ei[0], SENT) for ei in edges]
        + [shape_idx(ei[1], SENT) for ei in edges]
    )
    ones128 = jnp.ones((CHUNK, 128), jnp.float32)
    z128 = jnp.zeros((CHUNK, 128), jnp.float32)

    (deg,) = _hist_kernel()(hidx, ones128, z128)
    scales = _scales(deg)

    p = params
    y, st = _mm_stats(x, p['W_fr'], p['b_fr'].reshape(1, D))
    gb = jnp.stack([p['g_fr'], p['be_fr']])
    h_and_tables = _norm(y, st, gb, scales, leaky=False, emit_tables=True)
    h, tables = h_and_tables[0], h_and_tables[1:]

    n_layers = len(p['layers'])
    for li, lp in enumerate(p['layers']):
        aggs = _agg_kernel()(*tables, srcg, dstg, z128)
        wcat = jnp.concatenate(
            [w / ETYPES for w in lp['W_gcn']] + [lp['W_sk']], axis=0)
        bsum = (sum(lp['b_gcn']) / ETYPES + lp['b_sk']).reshape(1, D)
        y, st = _combine(aggs, scales, h, wcat, bsum)
        gb = jnp.stack([lp['g'], lp['be']])
        last = li == n_layers - 1
        res = _norm(y, st, gb, scales, leaky=True, emit_tables=not last)
        h, tables = res[0], res[1:]
    return h
